# baseline (speedup 1.0000x reference)
import functools

import jax
import jax.numpy as jnp
from jax import lax
from jax.experimental import pallas as pl
from jax.experimental.pallas import tpu as pltpu
from jax.experimental.pallas import tpu_sc as plsc

NC, NS, L = 2, 16, 16
NW = NC * NS
T = 128
RPC = 3 * T

Q = 4
GRP = Q * T


def _relayout_body(x0, x1, x2, x3, o_ref):
    for j, x in enumerate((x0, x1, x2, x3)):
        o_ref[:, 32 * j:32 * (j + 1)] = x[...].T


def _tc_relayout(tableT, vocab, embed):
    grid = (vocab + GRP - 1) // GRP
    last = (vocab + T - 1) // T - 1
    in_specs = [
        pl.BlockSpec(
            (embed, T),
            functools.partial(lambda j, i: (0, jnp.minimum(Q * i + j, last)), j),
        )
        for j in range(Q)
    ]
    return pl.pallas_call(
        _relayout_body,
        grid=(grid,),
        in_specs=in_specs,
        out_specs=pl.BlockSpec((T, Q * embed), lambda i: (i, 0)),
        out_shape=jax.ShapeDtypeStruct((grid * T, Q * embed), jnp.float32),
    )(tableT, tableT, tableT, tableT)


def _make_sc_call(B, S, M, vocab, embed, vrows):
    assert embed == 2 * L
    chunks = B * S * M // T
    assert chunks % NW == 0
    cpw = chunks // NW
    assert cpw % 2 == 0
    bblk = B // T

    mesh = plsc.VectorSubcoreMesh(core_axis_name="c", subcore_axis_name="s")

    @functools.partial(
        pl.kernel,
        out_type=jax.ShapeDtypeStruct((B, S, M, embed), jnp.float32),
        mesh=mesh,
        scratch_types=[
            pltpu.VMEM((2, 3, T), jnp.int32),
            pltpu.VMEM((2, RPC, embed), jnp.float32),
            pltpu.VMEM((T, embed), jnp.float32),
            pltpu.VMEM((16, L), jnp.float32),
            pltpu.SemaphoreType.DMA,
            pltpu.SemaphoreType.DMA,
        ],
        compiler_params=pltpu.CompilerParams(use_tc_tiling_on_sc=False),
    )
    def sc_call(inputs_hbm, table_hbm, w_hbm, out_hbm, idx_v, rows_v, out_v, wv,
                gsem0, gsem1):
        wid = lax.axis_index("s") * NC + lax.axis_index("c")
        pltpu.sync_copy(w_hbm, wv)
        gsems = (gsem0, gsem1)

        a = [[wv[3 * q + p] for p in range(3)] for q in range(3)]
        b0v = [wv[9 + p] for p in range(3)]
        w1v = [wv[12 + p] for p in range(3)]
        b1v = wv[15]

        def coords(c):
            s = c // (M * bblk)
            rem = c - s * (M * bblk)
            return s, rem // bblk, rem - (rem // bblk) * bblk

        def stage(buf, c):
            s, m, bb = coords(c)
            pltpu.sync_copy(inputs_hbm.at[s, :, m, pl.ds(bb * T, T)],
                            idx_v.at[buf])
            for k in range(3):
                for v in range(T // L):
                    sl = pl.ds(v * L, L)
                    ix = idx_v[buf, k, sl]
                    idx_v[buf, k, sl] = (
                        4 * (((ix >> 9) << 7) + (ix & (T - 1))) + ((ix >> 7) & 3)
                    )

        def gather_descs(buf):
            return [
                pltpu.make_async_copy(
                    table_hbm.at[idx_v.at[buf, k]],
                    rows_v.at[buf].at[pl.ds(k * T, T)],
                    gsems[buf],
                )
                for k in range(3)
            ]

        def issue(buf):
            for cp in gather_descs(buf):
                cp.start()

        def drain(buf):
            for cp in gather_descs(buf):
                cp.wait()

        def process(buf, c):
            rb = rows_v.at[buf]

            @plsc.parallel_loop(0, T, unroll=4)
            def _(t):
                for v in range(2):
                    sl = pl.ds(v * L, L)
                    e0 = rb[t, sl]
                    e1 = rb[T + t, sl]
                    e2 = rb[2 * T + t, sl]
                    o = b1v
                    for p in range(3):
                        h = e0 * a[0][p] + e1 * a[1][p] + e2 * a[2][p] + b0v[p]
                        h = jnp.maximum(h, 0.0)
                        o = o + h * w1v[p]
                    out_v[t, sl] = jnp.maximum(o, 0.0)

            s, m, bb = coords(c)
            pltpu.sync_copy(out_v, out_hbm.at[pl.ds(bb * T, T), s, m])

        stage(0, wid * cpw)
        issue(0)

        def pair_body(j, carry):
            c0 = wid * cpw + 2 * j
            stage(1, c0 + 1)
            issue(1)
            drain(0)
            process(0, c0)

            @pl.when(2 * j + 2 < cpw)
            def _():
                stage(0, c0 + 2)
                issue(0)

            drain(1)
            process(1, c0 + 1)
            return carry

        lax.fori_loop(0, cpw // 2, pair_body, 0)

    return sc_call


def kernel(inputs, table, W0, b0, W1, b1):
    B, S, M, P = inputs.shape
    vocab, embed = table.shape
    assert P == 3
    table128 = _tc_relayout(table.T, vocab, embed)
    vrows = table128.shape[0] * Q
    tbl_lin = table128.reshape(vrows, embed)

    A = W0 + jnp.eye(P, dtype=W0.dtype)
    wflat = jnp.concatenate([A.reshape(-1), b0, W1.reshape(-1), b1])
    wvec = jnp.broadcast_to(wflat[:, None], (16, L)).astype(jnp.float32)

    inputs_std = inputs.astype(jnp.int32).transpose(1, 3, 2, 0)
    return _make_sc_call(B, S, M, vocab, embed, vrows)(
        inputs_std, tbl_lin, wvec)


# speedup vs baseline: 1.9133x; 1.9133x over previous
import functools

import jax
import jax.numpy as jnp
from jax import lax
from jax.experimental import pallas as pl
from jax.experimental.pallas import tpu as pltpu
from jax.experimental.pallas import tpu_sc as plsc

NC, NS, L = 2, 16, 16
NW = NC * NS
T = 128
RPC = 3 * T


def _make_sc_call(B, S, M, vocab, embed):
    assert embed == 2 * L
    chunks = B * S * M // T
    assert chunks % NW == 0
    cpw = chunks // NW
    assert cpw % 2 == 0
    bblk = B // T

    mesh = plsc.VectorSubcoreMesh(core_axis_name="c", subcore_axis_name="s")

    @functools.partial(
        pl.kernel,
        out_type=jax.ShapeDtypeStruct((B, S, M, embed), jnp.float32),
        mesh=mesh,
        scratch_types=[
            pltpu.VMEM((2, 3, T), jnp.int32),
            pltpu.VMEM((2, RPC, embed), jnp.float32),
            pltpu.VMEM((T, embed), jnp.float32),
            pltpu.VMEM((16, L), jnp.float32),
            pltpu.SemaphoreType.DMA,
            pltpu.SemaphoreType.DMA,
        ],
        compiler_params=pltpu.CompilerParams(use_tc_tiling_on_sc=False),
    )
    def sc_call(inputs_hbm, table_hbm, w_hbm, out_hbm, idx_v, rows_v, out_v, wv,
                gsem0, gsem1):
        wid = lax.axis_index("s") * NC + lax.axis_index("c")
        pltpu.sync_copy(w_hbm, wv)
        gsems = (gsem0, gsem1)

        a = [[wv[3 * q + p] for p in range(3)] for q in range(3)]
        b0v = [wv[9 + p] for p in range(3)]
        w1v = [wv[12 + p] for p in range(3)]
        b1v = wv[15]

        def coords(c):
            s = c // (M * bblk)
            rem = c - s * (M * bblk)
            return s, rem // bblk, rem - (rem // bblk) * bblk

        def stage(buf, c):
            s, m, bb = coords(c)
            pltpu.sync_copy(inputs_hbm.at[s, :, m, pl.ds(bb * T, T)],
                            idx_v.at[buf])

        def gather_descs(buf):
            return [
                pltpu.make_async_copy(
                    table_hbm.at[idx_v.at[buf, k]],
                    rows_v.at[buf].at[pl.ds(k * T, T)],
                    gsems[buf],
                )
                for k in range(3)
            ]

        def issue(buf):
            for cp in gather_descs(buf):
                cp.start()

        def drain(buf):
            for cp in gather_descs(buf):
                cp.wait()

        def process(buf, c):
            rb = rows_v.at[buf]

            @plsc.parallel_loop(0, T, unroll=4)
            def _(t):
                for v in range(2):
                    sl = pl.ds(v * L, L)
                    e0 = rb[t, sl]
                    e1 = rb[T + t, sl]
                    e2 = rb[2 * T + t, sl]
                    o = b1v
                    for p in range(3):
                        h = e0 * a[0][p] + e1 * a[1][p] + e2 * a[2][p] + b0v[p]
                        h = jnp.maximum(h, 0.0)
                        o = o + h * w1v[p]
                    out_v[t, sl] = jnp.maximum(o, 0.0)

            s, m, bb = coords(c)
            pltpu.sync_copy(out_v, out_hbm.at[pl.ds(bb * T, T), s, m])

        stage(0, wid * cpw)
        issue(0)

        def pair_body(j, carry):
            c0 = wid * cpw + 2 * j
            stage(1, c0 + 1)
            issue(1)
            drain(0)
            process(0, c0)

            @pl.when(2 * j + 2 < cpw)
            def _():
                stage(0, c0 + 2)
                issue(0)

            drain(1)
            process(1, c0 + 1)
            return carry

        lax.fori_loop(0, cpw // 2, pair_body, 0)

    return sc_call


def kernel(inputs, table, W0, b0, W1, b1):
    B, S, M, P = inputs.shape
    vocab, embed = table.shape
    assert P == 3

    A = W0 + jnp.eye(P, dtype=W0.dtype)
    wflat = jnp.concatenate([A.reshape(-1), b0, W1.reshape(-1), b1])
    wvec = jnp.broadcast_to(wflat[:, None], (16, L)).astype(jnp.float32)

    inputs_std = inputs.astype(jnp.int32).transpose(1, 3, 2, 0)
    return _make_sc_call(B, S, M, vocab, embed)(
        inputs_std, table.astype(jnp.float32), wvec)


# speedup vs baseline: 1.9621x; 1.0255x over previous
import functools

import jax
import jax.numpy as jnp
from jax import lax
from jax.experimental import pallas as pl
from jax.experimental.pallas import tpu as pltpu
from jax.experimental.pallas import tpu_sc as plsc

NC, NS, L = 2, 16, 16
NW = NC * NS
T = 128
RPC = 3 * T


def _make_sc_call(B, S, M, vocab, embed):
    assert embed == 2 * L
    chunks = B * S * M // T
    assert chunks % NW == 0
    cpw = chunks // NW
    bblk = B // T

    mesh = plsc.VectorSubcoreMesh(core_axis_name="c", subcore_axis_name="s")

    @functools.partial(
        pl.kernel,
        out_type=jax.ShapeDtypeStruct((B, S, M, embed), jnp.float32),
        mesh=mesh,
        scratch_types=[
            pltpu.VMEM((2, 3, T), jnp.int32),
            pltpu.VMEM((2, RPC, embed), jnp.float32),
            pltpu.VMEM((T, embed), jnp.float32),
            pltpu.VMEM((16, L), jnp.float32),
            pltpu.SemaphoreType.DMA,
            pltpu.SemaphoreType.DMA,
        ],
        compiler_params=pltpu.CompilerParams(use_tc_tiling_on_sc=False),
    )
    def sc_call(inputs_hbm, table_hbm, w_hbm, out_hbm, idx_v, rows_v, out_v, wv,
                gsem0, gsem1):
        wid = lax.axis_index("s") * NC + lax.axis_index("c")
        pltpu.sync_copy(w_hbm, wv)
        gsems = (gsem0, gsem1)

        a = [[wv[3 * q + p] for p in range(3)] for q in range(3)]
        b0v = [wv[9 + p] for p in range(3)]
        w1v = [wv[12 + p] for p in range(3)]
        b1v = wv[15]

        def coords(c):
            s = c // (M * bblk)
            rem = c - s * (M * bblk)
            return s, rem // bblk, rem - (rem // bblk) * bblk

        def stage(buf, c):
            s, m, bb = coords(c)
            pltpu.sync_copy(inputs_hbm.at[s, :, m, pl.ds(bb * T, T)],
                            idx_v.at[buf])

        def gather_descs(buf):
            return [
                pltpu.make_async_copy(
                    table_hbm.at[idx_v.at[buf, k]],
                    rows_v.at[buf].at[pl.ds(k * T, T)],
                    gsems[buf],
                )
                for k in range(3)
            ]

        def issue(buf):
            for cp in gather_descs(buf):
                cp.start()

        def drain(buf):
            for cp in gather_descs(buf):
                cp.wait()

        def process(buf, c):
            rb = rows_v.at[buf]

            @plsc.parallel_loop(0, T, unroll=4)
            def _(t):
                for v in range(2):
                    sl = pl.ds(v * L, L)
                    e0 = rb[t, sl]
                    e1 = rb[T + t, sl]
                    e2 = rb[2 * T + t, sl]
                    o = b1v
                    for p in range(3):
                        h = e0 * a[0][p] + e1 * a[1][p] + e2 * a[2][p] + b0v[p]
                        h = jnp.maximum(h, 0.0)
                        o = o + h * w1v[p]
                    out_v[t, sl] = jnp.maximum(o, 0.0)

            s, m, bb = coords(c)
            pltpu.sync_copy(out_v, out_hbm.at[pl.ds(bb * T, T), s, m])

        stage(0, wid * cpw)
        issue(0)

        def pair_body(j, carry):
            c0 = wid * cpw + 2 * j
            stage(1, c0 + 1)
            issue(1)
            drain(0)
            process(0, c0)

            @pl.when(2 * j + 2 < cpw)
            def _():
                stage(0, c0 + 2)
                issue(0)

            drain(1)
            process(1, c0 + 1)
            return carry

        lax.fori_loop(0, cpw // 2, pair_body, 0)
        if cpw % 2:
            drain(0)
            process(0, wid * cpw + cpw - 1)

    return sc_call


def kernel(inputs, table, W0, b0, W1, b1):
    B, S, M, P = inputs.shape
    vocab, embed = table.shape
    assert P == 3

    A = W0 + jnp.eye(P, dtype=W0.dtype)
    wflat = jnp.concatenate([A.reshape(-1), b0, W1.reshape(-1), b1])
    wvec = jnp.broadcast_to(wflat[:, None], (16, L)).astype(jnp.float32)

    inputs_std = inputs.astype(jnp.int32).transpose(1, 3, 2, 0)
    tbl = table.astype(jnp.float32)

    H = B // 2
    call = _make_sc_call(H, S, M, vocab, embed)
    o1 = call(inputs_std[:, :, :, :H], tbl, wvec)
    o2 = call(inputs_std[:, :, :, H:], tbl, wvec)
    return jnp.concatenate([o1, o2], axis=0)


# speedup vs baseline: 1.9661x; 1.0020x over previous
import functools

import jax
import jax.numpy as jnp
from jax import lax
from jax.experimental import pallas as pl
from jax.experimental.pallas import tpu as pltpu
from jax.experimental.pallas import tpu_sc as plsc

NC, NS, L = 2, 16, 16
NW = NC * NS
T = 128
RPC = 3 * T


def _make_sc_call(B, S, M, vocab, embed):
    assert embed == 2 * L
    chunks = B * S * M // T
    assert chunks % NW == 0
    cpw = chunks // NW
    bblk = B // T

    mesh = plsc.VectorSubcoreMesh(core_axis_name="c", subcore_axis_name="s")

    @functools.partial(
        pl.kernel,
        out_type=jax.ShapeDtypeStruct((B, S, M, embed), jnp.float32),
        mesh=mesh,
        scratch_types=[
            pltpu.VMEM((2, 3, T), jnp.int32),
            pltpu.VMEM((2, RPC, embed), jnp.float32),
            pltpu.VMEM((T, embed), jnp.float32),
            pltpu.VMEM((16, L), jnp.float32),
            pltpu.SemaphoreType.DMA,
            pltpu.SemaphoreType.DMA,
        ],
        compiler_params=pltpu.CompilerParams(use_tc_tiling_on_sc=False),
    )
    def sc_call(inputs_hbm, table_hbm, w_hbm, out_hbm, idx_v, rows_v, out_v, wv,
                gsem0, gsem1):
        wid = lax.axis_index("s") * NC + lax.axis_index("c")
        pltpu.sync_copy(w_hbm, wv)
        gsems = (gsem0, gsem1)

        a = [[wv[3 * q + p] for p in range(3)] for q in range(3)]
        b0v = [wv[9 + p] for p in range(3)]
        w1v = [wv[12 + p] for p in range(3)]
        b1v = wv[15]

        def coords(c):
            s = c // (M * bblk)
            rem = c - s * (M * bblk)
            return s, rem // bblk, rem - (rem // bblk) * bblk

        def stage(buf, c):
            s, m, bb = coords(c)
            pltpu.sync_copy(inputs_hbm.at[s, :, m, pl.ds(bb * T, T)],
                            idx_v.at[buf])

        def gather_descs(buf):
            return [
                pltpu.make_async_copy(
                    table_hbm.at[idx_v.at[buf, k]],
                    rows_v.at[buf].at[pl.ds(k * T, T)],
                    gsems[buf],
                )
                for k in range(3)
            ]

        def issue(buf):
            for cp in gather_descs(buf):
                cp.start()

        def drain(buf):
            for cp in gather_descs(buf):
                cp.wait()

        def process(buf, c):
            rb = rows_v.at[buf]

            @plsc.parallel_loop(0, T, unroll=4)
            def _(t):
                for v in range(2):
                    sl = pl.ds(v * L, L)
                    e0 = rb[t, sl]
                    e1 = rb[T + t, sl]
                    e2 = rb[2 * T + t, sl]
                    o = b1v
                    for p in range(3):
                        h = e0 * a[0][p] + e1 * a[1][p] + e2 * a[2][p] + b0v[p]
                        h = jnp.maximum(h, 0.0)
                        o = o + h * w1v[p]
                    out_v[t, sl] = jnp.maximum(o, 0.0)

            s, m, bb = coords(c)
            pltpu.sync_copy(out_v, out_hbm.at[pl.ds(bb * T, T), s, m])

        stage(0, wid * cpw)
        issue(0)

        def pair_body(j, carry):
            c0 = wid * cpw + 2 * j
            stage(1, c0 + 1)
            issue(1)
            drain(0)
            process(0, c0)

            @pl.when(2 * j + 2 < cpw)
            def _():
                stage(0, c0 + 2)
                issue(0)

            drain(1)
            process(1, c0 + 1)
            return carry

        lax.fori_loop(0, cpw // 2, pair_body, 0)
        if cpw % 2:
            drain(0)
            process(0, wid * cpw + cpw - 1)

    return sc_call


def kernel(inputs, table, W0, b0, W1, b1):
    B, S, M, P = inputs.shape
    vocab, embed = table.shape
    assert P == 3

    A = W0 + jnp.eye(P, dtype=W0.dtype)
    wflat = jnp.concatenate([A.reshape(-1), b0, W1.reshape(-1), b1])
    wvec = jnp.broadcast_to(wflat[:, None], (16, L)).astype(jnp.float32)

    inputs_std = inputs.astype(jnp.int32).transpose(1, 3, 2, 0)
    tbl = table.astype(jnp.float32)

    H = S // 2
    call = _make_sc_call(B, H, M, vocab, embed)
    o1 = call(inputs_std[:H], tbl, wvec)
    o2 = call(inputs_std[H:], tbl, wvec)
    return jnp.concatenate([o1, o2], axis=1)
